# Initial kernel scaffold; baseline (speedup 1.0000x reference)
import jax
import jax.numpy as jnp
from jax.experimental import pallas as pl


def kernel(x, edge_index, batch, W1_0, b1_0, gamma_0, beta_0, W2_0, b2_0, W1_1, b1_1, gamma_1, beta_1, W2_1, b2_1, W1_2, b1_2, gamma_2, beta_2, W2_2, b2_2, lin1_W, lin1_b, lin2_W, lin2_b):
    raise NotImplementedError("write your pallas kernel here")


# speedup vs baseline: 2.6202x; 2.6202x over previous
import functools

import jax
import jax.numpy as jnp
from jax import lax
from jax.experimental import pallas as pl
from jax.experimental.pallas import tpu as pltpu
from jax.experimental.pallas import tpu_sc as plsc

N = 10000
E = 320000
D = 128
H = 128
G = 128
C = 10

NC = 2
NS = 16
NW = NC * NS

CH = 128
K = 80
EPW = CH * K
E_PAD = EPW * NW
ACC_ROWS = 10240
PAD_DST = 10112
ZROWS_PER_SUB = ACC_ROWS // NS
WROWS_PER_SUB = 624
WTAIL = N - NS * WROWS_PER_SUB

_mesh = plsc.VectorSubcoreMesh(core_axis_name="c", subcore_axis_name="s")


@functools.partial(
    pl.kernel,
    mesh=_mesh,
    out_type=jax.ShapeDtypeStruct((NC, N, H), jnp.float32),
    scratch_types=[
        pltpu.VMEM((K, CH), jnp.int32),
        pltpu.VMEM((K, CH), jnp.int32),
        pltpu.VMEM((CH, H), jnp.float32),
        pltpu.VMEM_SHARED((ACC_ROWS, H), jnp.float32),
    ],
)
def _sc_aggregate(h_hbm, src_hbm, dst_hbm, out_hbm, sidx, didx, rows, acc):
    c = lax.axis_index("c")
    s = lax.axis_index("s")
    w = c * NS + s

    @pl.loop(0, CH)
    def _(i):
        @pl.loop(0, H // 16)
        def _(j):
            rows[i, pl.ds(j * 16, 16)] = jnp.zeros((16,), jnp.float32)

    @pl.loop(0, ZROWS_PER_SUB // CH)
    def _(t):
        pltpu.sync_copy(rows, acc.at[pl.ds(s * ZROWS_PER_SUB + t * CH, CH)])

    pltpu.sync_copy(src_hbm.at[w], sidx)
    pltpu.sync_copy(dst_hbm.at[w], didx)

    plsc.subcore_barrier()

    @pl.loop(0, K)
    def _(k):
        pltpu.sync_copy(h_hbm.at[sidx.at[k]], rows)
        pltpu.sync_copy(rows, acc.at[didx.at[k]], add=True)

    plsc.subcore_barrier()

    pltpu.sync_copy(
        acc.at[pl.ds(s * WROWS_PER_SUB, WROWS_PER_SUB)],
        out_hbm.at[c, pl.ds(s * WROWS_PER_SUB, WROWS_PER_SUB)],
    )

    @pl.when(s == 0)
    def _():
        pltpu.sync_copy(
            acc.at[pl.ds(NS * WROWS_PER_SUB, WTAIL)],
            out_hbm.at[c, pl.ds(NS * WROWS_PER_SUB, WTAIL)],
        )


BLK = 2000
NBLK = N // BLK


def _mm1_body(h_ref, p0_ref, p1_ref, w1_ref, b1_ref, z_ref, ssum_ref, ssq_ref):
    i = pl.program_id(0)
    z = h_ref[...] + p0_ref[...] + p1_ref[...]
    z = jnp.dot(z, w1_ref[...], preferred_element_type=jnp.float32,
                precision=lax.Precision.HIGHEST) + b1_ref[...]
    z_ref[...] = z

    @pl.when(i == 0)
    def _():
        ssum_ref[...] = jnp.zeros_like(ssum_ref)
        ssq_ref[...] = jnp.zeros_like(ssq_ref)

    ssum_ref[...] += jnp.sum(z, axis=0, keepdims=True)
    ssq_ref[...] += jnp.sum(z * z, axis=0, keepdims=True)


_tc_mm1 = pl.pallas_call(
    _mm1_body,
    grid=(NBLK,),
    in_specs=[
        pl.BlockSpec((BLK, H), lambda i: (i, 0)),
        pl.BlockSpec((BLK, H), lambda i: (i, 0)),
        pl.BlockSpec((BLK, H), lambda i: (i, 0)),
        pl.BlockSpec((H, H), lambda i: (0, 0)),
        pl.BlockSpec((1, H), lambda i: (0, 0)),
    ],
    out_specs=[
        pl.BlockSpec((BLK, H), lambda i: (i, 0)),
        pl.BlockSpec((1, H), lambda i: (0, 0)),
        pl.BlockSpec((1, H), lambda i: (0, 0)),
    ],
    out_shape=[
        jax.ShapeDtypeStruct((N, H), jnp.float32),
        jax.ShapeDtypeStruct((1, H), jnp.float32),
        jax.ShapeDtypeStruct((1, H), jnp.float32),
    ],
)


def _mm2_body(z_ref, ssum_ref, ssq_ref, batch_ref, g_ref, be_ref, w2_ref,
              b2_ref, hout_ref, pool_ref):
    i = pl.program_id(0)
    mean = ssum_ref[...] * (1.0 / N)
    var = ssq_ref[...] * (1.0 / N) - mean * mean
    z = (z_ref[...] - mean) * lax.rsqrt(var + 1e-5) * g_ref[...] + be_ref[...]
    z = jnp.maximum(z, 0.0)
    z = jnp.dot(z, w2_ref[...], preferred_element_type=jnp.float32,
                precision=lax.Precision.HIGHEST) + b2_ref[...]
    z = jnp.maximum(z, 0.0)
    hout_ref[...] = z
    gids = lax.broadcasted_iota(jnp.int32, (G, BLK), 0)
    sel = (batch_ref[0] == gids).astype(jnp.float32)

    @pl.when(i == 0)
    def _():
        pool_ref[...] = jnp.zeros_like(pool_ref)

    pool_ref[...] += jnp.dot(sel, z, preferred_element_type=jnp.float32,
                             precision=lax.Precision.HIGHEST)


_tc_mm2 = pl.pallas_call(
    _mm2_body,
    grid=(NBLK,),
    in_specs=[
        pl.BlockSpec((BLK, H), lambda i: (i, 0)),
        pl.BlockSpec((1, H), lambda i: (0, 0)),
        pl.BlockSpec((1, H), lambda i: (0, 0)),
        pl.BlockSpec((1, 1, BLK), lambda i: (i, 0, 0)),
        pl.BlockSpec((1, H), lambda i: (0, 0)),
        pl.BlockSpec((1, H), lambda i: (0, 0)),
        pl.BlockSpec((H, H), lambda i: (0, 0)),
        pl.BlockSpec((1, H), lambda i: (0, 0)),
    ],
    out_specs=[
        pl.BlockSpec((BLK, H), lambda i: (i, 0)),
        pl.BlockSpec((G, H), lambda i: (0, 0)),
    ],
    out_shape=[
        jax.ShapeDtypeStruct((N, H), jnp.float32),
        jax.ShapeDtypeStruct((G, H), jnp.float32),
    ],
)


def _readout_body(p0_ref, p1_ref, p2_ref, w1_ref, b1_ref, w2_ref, b2_ref,
                  out_ref):
    hg = jnp.concatenate([p0_ref[...], p1_ref[...], p2_ref[...]], axis=1)
    hg = jnp.dot(hg, w1_ref[...], preferred_element_type=jnp.float32,
                 precision=lax.Precision.HIGHEST) + b1_ref[...]
    hg = jnp.maximum(hg, 0.0)
    out_ref[...] = jnp.dot(hg, w2_ref[...], preferred_element_type=jnp.float32,
                           precision=lax.Precision.HIGHEST) + b2_ref[...]


_tc_readout = pl.pallas_call(
    _readout_body,
    out_shape=jax.ShapeDtypeStruct((G, H), jnp.float32),
)


def kernel(x, edge_index, batch,
           W1_0, b1_0, gamma_0, beta_0, W2_0, b2_0,
           W1_1, b1_1, gamma_1, beta_1, W2_1, b2_1,
           W1_2, b1_2, gamma_2, beta_2, W2_2, b2_2,
           lin1_W, lin1_b, lin2_W, lin2_b):
    src = edge_index[0]
    dst = edge_index[1]
    pad = E_PAD - E
    src3 = jnp.concatenate([src, jnp.zeros((pad,), jnp.int32)]).reshape(NW, K, CH)
    dst3 = jnp.concatenate([dst, jnp.full((pad,), PAD_DST, jnp.int32)]).reshape(NW, K, CH)
    batch3 = batch.reshape(NBLK, 1, BLK)

    layer_params = [
        (W1_0, b1_0, gamma_0, beta_0, W2_0, b2_0),
        (W1_1, b1_1, gamma_1, beta_1, W2_1, b2_1),
        (W1_2, b1_2, gamma_2, beta_2, W2_2, b2_2),
    ]

    h = x
    pools = []
    for (w1, b1, gamma, beta, w2, b2) in layer_params:
        parts = _sc_aggregate(h, src3, dst3)
        z, ssum, ssq = _tc_mm1(h, parts[0], parts[1], w1, b1.reshape(1, H))
        h, pool = _tc_mm2(z, ssum, ssq, batch3, gamma.reshape(1, H),
                          beta.reshape(1, H), w2, b2.reshape(1, H))
        pools.append(pool)

    lin2_Wp = jnp.pad(lin2_W, ((0, 0), (0, H - C)))
    lin2_bp = jnp.pad(lin2_b, (0, H - C)).reshape(1, H)
    out = _tc_readout(pools[0], pools[1], pools[2],
                      lin1_W, lin1_b.reshape(1, H * 3), lin2_Wp, lin2_bp)
    return out[:, :C]


# speedup vs baseline: 2.9246x; 1.1162x over previous
import functools

import jax
import jax.numpy as jnp
from jax import lax
from jax.experimental import pallas as pl
from jax.experimental.pallas import tpu as pltpu
from jax.experimental.pallas import tpu_sc as plsc

N = 10000
E = 320000
D = 128
H = 128
G = 128
C = 10

NC = 2
NS = 16
NW = NC * NS

CH = 128
K = 80
K2 = K // 2
EPW = CH * K
E_PAD = EPW * NW
ACC_ROWS = 10240
PAD_DST = 10112
ZROWS_PER_SUB = ACC_ROWS // NS
WROWS_PER_SUB = 624
WTAIL = N - NS * WROWS_PER_SUB

_mesh = plsc.VectorSubcoreMesh(core_axis_name="c", subcore_axis_name="s")

NBUF = 2


@functools.partial(
    pl.kernel,
    mesh=_mesh,
    out_type=jax.ShapeDtypeStruct((NC, N, H), jnp.float32),
    scratch_types=[
        pltpu.VMEM((K2, CH), jnp.int32),
        pltpu.VMEM((K2, CH), jnp.int32),
        pltpu.VMEM((NBUF, CH, H), jnp.float32),
        pltpu.SemaphoreType.DMA,
        pltpu.SemaphoreType.DMA,
        pltpu.VMEM_SHARED((ACC_ROWS, H), jnp.float32),
    ],
)
def _sc_aggregate(h_hbm, src_hbm, dst_hbm, out_hbm, sidx, didx, rows,
                  sem0, sem1, acc):
    c = lax.axis_index("c")
    s = lax.axis_index("s")
    w = c * NS + s
    sems = (sem0, sem1)

    @pl.loop(0, CH)
    def _(i):
        @pl.loop(0, H // 16)
        def _(j):
            rows[0, i, pl.ds(j * 16, 16)] = jnp.zeros((16,), jnp.float32)

    @pl.loop(0, ZROWS_PER_SUB // CH)
    def _(t):
        pltpu.sync_copy(rows.at[0], acc.at[pl.ds(s * ZROWS_PER_SUB + t * CH, CH)])

    plsc.subcore_barrier()

    @pl.loop(0, 2)
    def _(hh):
        pltpu.sync_copy(src_hbm.at[w, pl.ds(hh * K2, K2)], sidx)
        pltpu.sync_copy(dst_hbm.at[w, pl.ds(hh * K2, K2)], didx)

        for b in range(NBUF):
            pltpu.async_copy(h_hbm.at[sidx.at[b]], rows.at[b], sems[b])

        @pl.loop(0, K2, step=NBUF)
        def _(k):
            for b in range(NBUF):
                pltpu.make_async_copy(h_hbm.at[sidx.at[b]], rows.at[b],
                                      sems[b]).wait()
                pltpu.sync_copy(rows.at[b], acc.at[didx.at[k + b]], add=True)

                @pl.when(k + b + NBUF < K2)
                def _():
                    pltpu.async_copy(h_hbm.at[sidx.at[k + b + NBUF]],
                                     rows.at[b], sems[b])

    plsc.subcore_barrier()

    pltpu.sync_copy(
        acc.at[pl.ds(s * WROWS_PER_SUB, WROWS_PER_SUB)],
        out_hbm.at[c, pl.ds(s * WROWS_PER_SUB, WROWS_PER_SUB)],
    )

    @pl.when(s == 0)
    def _():
        pltpu.sync_copy(
            acc.at[pl.ds(NS * WROWS_PER_SUB, WTAIL)],
            out_hbm.at[c, pl.ds(NS * WROWS_PER_SUB, WTAIL)],
        )


BLK = 2000
NBLK = N // BLK


def _mm1_body(h_ref, p0_ref, p1_ref, w1_ref, b1_ref, z_ref, ssum_ref, ssq_ref):
    i = pl.program_id(0)
    z = h_ref[...] + p0_ref[...] + p1_ref[...]
    z = jnp.dot(z, w1_ref[...], preferred_element_type=jnp.float32,
                precision=lax.Precision.HIGHEST) + b1_ref[...]
    z_ref[...] = z

    @pl.when(i == 0)
    def _():
        ssum_ref[...] = jnp.zeros_like(ssum_ref)
        ssq_ref[...] = jnp.zeros_like(ssq_ref)

    ssum_ref[...] += jnp.sum(z, axis=0, keepdims=True)
    ssq_ref[...] += jnp.sum(z * z, axis=0, keepdims=True)


_tc_mm1 = pl.pallas_call(
    _mm1_body,
    grid=(NBLK,),
    in_specs=[
        pl.BlockSpec((BLK, H), lambda i: (i, 0)),
        pl.BlockSpec((BLK, H), lambda i: (i, 0)),
        pl.BlockSpec((BLK, H), lambda i: (i, 0)),
        pl.BlockSpec((H, H), lambda i: (0, 0)),
        pl.BlockSpec((1, H), lambda i: (0, 0)),
    ],
    out_specs=[
        pl.BlockSpec((BLK, H), lambda i: (i, 0)),
        pl.BlockSpec((1, H), lambda i: (0, 0)),
        pl.BlockSpec((1, H), lambda i: (0, 0)),
    ],
    out_shape=[
        jax.ShapeDtypeStruct((N, H), jnp.float32),
        jax.ShapeDtypeStruct((1, H), jnp.float32),
        jax.ShapeDtypeStruct((1, H), jnp.float32),
    ],
)


def _mm2_body(z_ref, ssum_ref, ssq_ref, batch_ref, g_ref, be_ref, w2_ref,
              b2_ref, hout_ref, pool_ref):
    i = pl.program_id(0)
    mean = ssum_ref[...] * (1.0 / N)
    var = ssq_ref[...] * (1.0 / N) - mean * mean
    z = (z_ref[...] - mean) * lax.rsqrt(var + 1e-5) * g_ref[...] + be_ref[...]
    z = jnp.maximum(z, 0.0)
    z = jnp.dot(z, w2_ref[...], preferred_element_type=jnp.float32,
                precision=lax.Precision.HIGHEST) + b2_ref[...]
    z = jnp.maximum(z, 0.0)
    hout_ref[...] = z
    gids = lax.broadcasted_iota(jnp.int32, (G, BLK), 0)
    sel = (batch_ref[0] == gids).astype(jnp.float32)

    @pl.when(i == 0)
    def _():
        pool_ref[...] = jnp.zeros_like(pool_ref)

    pool_ref[...] += jnp.dot(sel, z, preferred_element_type=jnp.float32,
                             precision=lax.Precision.HIGHEST)


_tc_mm2 = pl.pallas_call(
    _mm2_body,
    grid=(NBLK,),
    in_specs=[
        pl.BlockSpec((BLK, H), lambda i: (i, 0)),
        pl.BlockSpec((1, H), lambda i: (0, 0)),
        pl.BlockSpec((1, H), lambda i: (0, 0)),
        pl.BlockSpec((1, 1, BLK), lambda i: (i, 0, 0)),
        pl.BlockSpec((1, H), lambda i: (0, 0)),
        pl.BlockSpec((1, H), lambda i: (0, 0)),
        pl.BlockSpec((H, H), lambda i: (0, 0)),
        pl.BlockSpec((1, H), lambda i: (0, 0)),
    ],
    out_specs=[
        pl.BlockSpec((BLK, H), lambda i: (i, 0)),
        pl.BlockSpec((G, H), lambda i: (0, 0)),
    ],
    out_shape=[
        jax.ShapeDtypeStruct((N, H), jnp.float32),
        jax.ShapeDtypeStruct((G, H), jnp.float32),
    ],
)


def _readout_body(p0_ref, p1_ref, p2_ref, w1_ref, b1_ref, w2_ref, b2_ref,
                  out_ref):
    hg = jnp.concatenate([p0_ref[...], p1_ref[...], p2_ref[...]], axis=1)
    hg = jnp.dot(hg, w1_ref[...], preferred_element_type=jnp.float32,
                 precision=lax.Precision.HIGHEST) + b1_ref[...]
    hg = jnp.maximum(hg, 0.0)
    out_ref[...] = jnp.dot(hg, w2_ref[...], preferred_element_type=jnp.float32,
                           precision=lax.Precision.HIGHEST) + b2_ref[...]


_tc_readout = pl.pallas_call(
    _readout_body,
    out_shape=jax.ShapeDtypeStruct((G, H), jnp.float32),
)


def kernel(x, edge_index, batch,
           W1_0, b1_0, gamma_0, beta_0, W2_0, b2_0,
           W1_1, b1_1, gamma_1, beta_1, W2_1, b2_1,
           W1_2, b1_2, gamma_2, beta_2, W2_2, b2_2,
           lin1_W, lin1_b, lin2_W, lin2_b):
    src = edge_index[0]
    dst = edge_index[1]
    pad = E_PAD - E
    src3 = jnp.concatenate([src, jnp.zeros((pad,), jnp.int32)]).reshape(NW, K, CH)
    dst3 = jnp.concatenate([dst, jnp.full((pad,), PAD_DST, jnp.int32)]).reshape(NW, K, CH)
    batch3 = batch.reshape(NBLK, 1, BLK)

    layer_params = [
        (W1_0, b1_0, gamma_0, beta_0, W2_0, b2_0),
        (W1_1, b1_1, gamma_1, beta_1, W2_1, b2_1),
        (W1_2, b1_2, gamma_2, beta_2, W2_2, b2_2),
    ]

    h = x
    pools = []
    for (w1, b1, gamma, beta, w2, b2) in layer_params:
        parts = _sc_aggregate(h, src3, dst3)
        z, ssum, ssq = _tc_mm1(h, parts[0], parts[1], w1, b1.reshape(1, H))
        h, pool = _tc_mm2(z, ssum, ssq, batch3, gamma.reshape(1, H),
                          beta.reshape(1, H), w2, b2.reshape(1, H))
        pools.append(pool)

    lin2_Wp = jnp.pad(lin2_W, ((0, 0), (0, H - C)))
    lin2_bp = jnp.pad(lin2_b, (0, H - C)).reshape(1, H)
    out = _tc_readout(pools[0], pools[1], pools[2],
                      lin1_W, lin1_b.reshape(1, H * 3), lin2_Wp, lin2_bp)
    return out[:, :C]
